# Initial kernel scaffold; baseline (speedup 1.0000x reference)
import functools

import jax
import jax.numpy as jnp
from jax import lax
from jax.experimental import pallas as pl
from jax.experimental.pallas import tpu as pltpu, tpu_sc as plsc

USER_N = 25000
ITEM_N = 25000
NODES = USER_N + ITEM_N
EMB = 64
HALF = EMB // 2
N_EDGES = 800000
N_LAYERS = 3

TILES = 16
CHUNK = 128
KCH = 4
BATCH = KCH * CHUNK
NB = 98
NE_PAD = TILES * NB * BATCH
ROWS_PER_TILE = NODES // TILES


def _spmm_body(x_hbm, col_hbm, dst_hbm, val_hbm, y_hbm,
               colbuf, dstbuf, vals, rows, acc, sem):
    c = lax.axis_index("c")
    s = lax.axis_index("s")

    def _zero(i, _):
        z = jnp.zeros((16,), jnp.float32)
        rows[i, pl.ds(0, 16)] = z
        rows[i, pl.ds(16, 16)] = z
        return 0
    lax.fori_loop(0, BATCH, _zero, 0)
    base_row = s * ROWS_PER_TILE
    for t in range(ROWS_PER_TILE // BATCH):
        pltpu.sync_copy(rows, acc.at[pl.ds(base_row + t * BATCH, BATCH)])
    rem = ROWS_PER_TILE % BATCH
    pltpu.sync_copy(rows.at[pl.ds(0, rem)],
                    acc.at[pl.ds(base_row + ROWS_PER_TILE - rem, rem)])
    plsc.subcore_barrier()

    col_off = c * NODES

    def _batch(b, _):
        bi = s * NB + b
        pltpu.sync_copy(col_hbm.at[bi], colbuf)
        pltpu.sync_copy(dst_hbm.at[bi], dstbuf)
        pltpu.sync_copy(val_hbm.at[bi], vals)
        for j in range(KCH):
            def _adj(k, _, j=j):
                cv = colbuf[j, pl.ds(k * 16, 16)]
                colbuf[j, pl.ds(k * 16, 16)] = cv + col_off
                return 0
            lax.fori_loop(0, CHUNK // 16, _adj, 0)
        handles = [
            pltpu.async_copy(x_hbm.at[colbuf.at[j]],
                             rows.at[pl.ds(j * CHUNK, CHUNK)], sem)
            for j in range(KCH)
        ]
        for h in handles:
            h.wait()
        def _scale(g, _):
            e0 = g * 16
            for k in range(16):
                e = e0 + k
                v = vals[e]
                rows[e, pl.ds(0, 16)] = rows[e, pl.ds(0, 16)] * v
                rows[e, pl.ds(16, 16)] = rows[e, pl.ds(16, 16)] * v
            return 0
        lax.fori_loop(0, BATCH // 16, _scale, 0)
        for j in range(KCH):
            pltpu.sync_copy(rows.at[pl.ds(j * CHUNK, CHUNK)],
                            acc.at[dstbuf.at[j]], add=True)
        return 0

    lax.fori_loop(0, NB, _batch, 0)
    plsc.subcore_barrier()
    pltpu.sync_copy(acc.at[pl.ds(base_row, ROWS_PER_TILE)],
                    y_hbm.at[pl.ds(col_off + base_row, ROWS_PER_TILE)])


@functools.partial(jax.jit, static_argnames=("interpret",))
def _spmm(x, colp, dstp, valp, interpret=False):
    mesh = plsc.VectorSubcoreMesh(core_axis_name="c", subcore_axis_name="s")
    return pl.kernel(
        _spmm_body,
        out_type=jax.ShapeDtypeStruct((2 * NODES, HALF), jnp.float32),
        mesh=mesh,
        scratch_types=[
            pltpu.VMEM((KCH, CHUNK), jnp.int32),
            pltpu.VMEM((KCH, CHUNK), jnp.int32),
            pltpu.SMEM((BATCH,), jnp.float32),
            pltpu.VMEM((BATCH, HALF), jnp.float32),
            pltpu.VMEM_SHARED((NODES, HALF), jnp.float32),
            pltpu.SemaphoreType.DMA,
        ],
        interpret=interpret,
    )(x, colp, dstp, valp)


def kernel(rec_user_emb, rec_item_emb, adj_row, adj_col, adj_val,
           interpret=False):
    ego = jnp.concatenate([rec_user_emb, rec_item_emb], axis=0)
    x = jnp.concatenate([ego[:, :HALF], ego[:, HALF:]], axis=0)
    pad = NE_PAD - N_EDGES
    colp = jnp.concatenate(
        [adj_col, jnp.zeros((pad,), jnp.int32)]).reshape(TILES * NB, KCH, CHUNK)
    dstp = jnp.concatenate(
        [adj_row, jnp.zeros((pad,), jnp.int32)]).reshape(TILES * NB, KCH, CHUNK)
    valp = jnp.concatenate(
        [adj_val, jnp.zeros((pad,), jnp.float32)]).reshape(TILES * NB, BATCH)

    xs = []
    for _ in range(N_LAYERS):
        x = _spmm(x, colp, dstp, valp, interpret=interpret)
        xs.append(x)
    m = (xs[0] + xs[1] + xs[2]) * (1.0 / N_LAYERS)
    out = jnp.concatenate([m[:NODES], m[NODES:]], axis=1)
    return out[:USER_N], out[USER_N:]


# speedup vs baseline: 6.7323x; 6.7323x over previous
import functools

import jax
import jax.numpy as jnp
from jax import lax
from jax.experimental import pallas as pl
from jax.experimental.pallas import tpu as pltpu, tpu_sc as plsc

USER_N = 25000
ITEM_N = 25000
NODES = USER_N + ITEM_N
EMB = 64
HALF = EMB // 2
N_EDGES = 800000
N_LAYERS = 3
NODESP = 50048

TILES = 16
CHUNK = 128
KCH = 4
BATCH = KCH * CHUNK
NB = 98
NE_PAD = TILES * NB * BATCH
ROWS_PER_TILE = NODESP // TILES


def _spmm_body(x_hbm, col_hbm, dst_hbm, val_hbm, y_hbm,
               colbuf, dstbuf, vals, rows, acc, sem):
    c = lax.axis_index("c")
    s = lax.axis_index("s")

    def _zero(i, _):
        z = jnp.zeros((16,), jnp.float32)
        rows[i, pl.ds(0, 16)] = z
        rows[i, pl.ds(16, 16)] = z
        return 0
    lax.fori_loop(0, BATCH, _zero, 0)
    base_row = s * ROWS_PER_TILE
    for t in range(ROWS_PER_TILE // BATCH):
        pltpu.sync_copy(rows, acc.at[pl.ds(base_row + t * BATCH, BATCH)])
    rem = ROWS_PER_TILE % BATCH
    pltpu.sync_copy(rows.at[pl.ds(0, rem)],
                    acc.at[pl.ds(base_row + ROWS_PER_TILE - rem, rem)])
    plsc.subcore_barrier()

    col_off = c * NODESP

    def _batch(b, _):
        bi = s * NB + b
        pltpu.sync_copy(col_hbm.at[bi], colbuf)
        pltpu.sync_copy(dst_hbm.at[bi], dstbuf)
        pltpu.sync_copy(val_hbm.at[bi], vals)
        for j in range(KCH):
            def _adj(k, _, j=j):
                cv = colbuf[j, pl.ds(k * 16, 16)]
                colbuf[j, pl.ds(k * 16, 16)] = cv + col_off
                return 0
            lax.fori_loop(0, CHUNK // 16, _adj, 0)
        handles = [
            pltpu.async_copy(x_hbm.at[colbuf.at[j]],
                             rows.at[pl.ds(j * CHUNK, CHUNK)], sem)
            for j in range(KCH)
        ]
        for h in handles:
            h.wait()
        for j in range(KCH):
            def _scale(g, _, j=j):
                vv = vals[j, pl.ds(g * 16, 16)]
                e0 = j * CHUNK + g * 16
                for k in range(16):
                    e = e0 + k
                    v = vv[k]
                    rows[e, pl.ds(0, 16)] = rows[e, pl.ds(0, 16)] * v
                    rows[e, pl.ds(16, 16)] = rows[e, pl.ds(16, 16)] * v
                return 0
            lax.fori_loop(0, CHUNK // 16, _scale, 0)
        for j in range(KCH):
            pltpu.sync_copy(rows.at[pl.ds(j * CHUNK, CHUNK)],
                            acc.at[dstbuf.at[j]], add=True)
        return 0

    lax.fori_loop(0, NB, _batch, 0)
    plsc.subcore_barrier()
    pltpu.sync_copy(acc.at[pl.ds(base_row, ROWS_PER_TILE)],
                    y_hbm.at[pl.ds(col_off + base_row, ROWS_PER_TILE)])


@functools.partial(jax.jit, static_argnames=("interpret",))
def _spmm(x, colp, dstp, valp, interpret=False):
    mesh = plsc.VectorSubcoreMesh(core_axis_name="c", subcore_axis_name="s",
                                  num_cores=2, num_subcores=TILES)
    return pl.kernel(
        _spmm_body,
        out_type=jax.ShapeDtypeStruct((2 * NODESP, HALF), jnp.float32),
        mesh=mesh,
        scratch_types=[
            pltpu.VMEM((KCH, CHUNK), jnp.int32),
            pltpu.VMEM((KCH, CHUNK), jnp.int32),
            pltpu.VMEM((KCH, CHUNK), jnp.float32),
            pltpu.VMEM((BATCH, HALF), jnp.float32),
            pltpu.VMEM_SHARED((NODESP, HALF), jnp.float32),
            pltpu.SemaphoreType.DMA,
        ],
        compiler_params=pltpu.CompilerParams(use_tc_tiling_on_sc=False),
        interpret=interpret,
    )(x, colp, dstp, valp)


def kernel(rec_user_emb, rec_item_emb, adj_row, adj_col, adj_val,
           interpret=False):
    ego = jnp.concatenate([rec_user_emb, rec_item_emb], axis=0)
    zpad = jnp.zeros((NODESP - NODES, HALF), jnp.float32)
    x = jnp.concatenate([ego[:, :HALF], zpad, ego[:, HALF:], zpad], axis=0)
    pad = NE_PAD - N_EDGES
    colp = jnp.concatenate(
        [adj_col, jnp.zeros((pad,), jnp.int32)]).reshape(TILES * NB, KCH, CHUNK)
    dstp = jnp.concatenate(
        [adj_row, jnp.zeros((pad,), jnp.int32)]).reshape(TILES * NB, KCH, CHUNK)
    valp = jnp.concatenate(
        [adj_val, jnp.zeros((pad,), jnp.float32)]).reshape(TILES * NB, KCH, CHUNK)

    xs = []
    for _ in range(N_LAYERS):
        x = _spmm(x, colp, dstp, valp, interpret=interpret)
        xs.append(x)
    m = (xs[0] + xs[1] + xs[2]) * (1.0 / N_LAYERS)
    out = jnp.concatenate([m[:NODES], m[NODESP:NODESP + NODES]], axis=1)
    return out[:USER_N], out[USER_N:]
